# Initial kernel scaffold; baseline (speedup 1.0000x reference)
import jax
import jax.numpy as jnp
from jax.experimental import pallas as pl


def kernel(params_maps, center_map, batch_ids, flat_inds, meta_batch_ids):
    raise NotImplementedError("write your pallas kernel here")


# speedup vs baseline: 4.4534x; 4.4534x over previous
import functools

import jax
import jax.numpy as jnp
from jax import lax
from jax.experimental import pallas as pl
from jax.experimental.pallas import tpu as pltpu
from jax.experimental.pallas import tpu_sc as plsc

B = 16
C = 145
H = 128
W = 128
HW = H * W
N = 2048
CPAD = 160
NW = 32
NDET = N // NW
NCHUNK = NDET * CPAD // 128
LANES = 16


def _sc_body(pm_hbm, cm_hbm, bid_hbm, hw_hbm, meta_hbm,
             out_params, out_conf, out_px, out_py, out_reorg,
             idx2d, buf, bids_v, hw_v, base_v, cidx_v, conf_buf,
             meta_v, reorg_buf, px_buf, py_buf, sem, sem2):
    wid = lax.axis_index("s") * 2 + lax.axis_index("c")
    det0 = wid * NDET

    pltpu.sync_copy(bid_hbm.at[pl.ds(det0, NDET)], bids_v)
    pltpu.sync_copy(hw_hbm.at[pl.ds(det0, NDET)], hw_v)
    pltpu.sync_copy(meta_hbm, meta_v)

    for t in range(NDET // LANES):
        sl = pl.ds(t * LANES, LANES)
        b = bids_v[sl]
        hw = hw_v[sl]
        base_v[sl] = b * (C * HW) + hw
        cidx_v[sl] = b * HW + hw
        reorg_buf[sl] = plsc.load_gather(meta_v, [b])
        px_buf[sl] = (hw & (W - 1)).astype(jnp.float32) * 4.0
        py_buf[sl] = lax.shift_right_logical(hw, 7).astype(jnp.float32) * 4.0

    def gen(j, _):
        for v in range(8):
            p0 = pl.multiple_of(j * 128, 128) + v * LANES
            p = p0 + lax.iota(jnp.int32, LANES)
            n_loc = p // CPAD
            c = p - n_loc * CPAD
            c = jnp.minimum(c, C - 1)
            bse = plsc.load_gather(base_v, [n_loc])
            idx2d[j, pl.ds(v * LANES, LANES)] = bse + c * HW
        return 0

    lax.fori_loop(0, NCHUNK, gen, 0)

    def fire(j, _):
        pltpu.async_copy(pm_hbm.at[idx2d.at[j]],
                         buf.at[pl.ds(pl.multiple_of(j * 128, 128), 128)],
                         sem)
        return 0

    lax.fori_loop(0, NCHUNK, fire, 0)

    pltpu.async_copy(cm_hbm.at[cidx_v], conf_buf, sem2).wait()
    pltpu.sync_copy(conf_buf, out_conf.at[pl.ds(det0, NDET)])
    pltpu.sync_copy(px_buf, out_px.at[pl.ds(det0, NDET)])
    pltpu.sync_copy(py_buf, out_py.at[pl.ds(det0, NDET)])
    pltpu.sync_copy(reorg_buf, out_reorg.at[pl.ds(det0, NDET)])

    def drain(j, _):
        pltpu.make_async_copy(pm_hbm.at[idx2d.at[j]],
                              buf.at[pl.ds(pl.multiple_of(j * 128, 128), 128)],
                              sem).wait()
        return 0

    lax.fori_loop(0, NCHUNK, drain, 0)

    pltpu.sync_copy(buf, out_params.at[pl.ds(det0 * CPAD, NDET * CPAD)])


@jax.jit
def kernel(params_maps, center_map, batch_ids, flat_inds, meta_batch_ids):
    pm_flat = params_maps.reshape(-1)
    cm_flat = center_map.reshape(-1)

    mesh = plsc.VectorSubcoreMesh(core_axis_name="c", subcore_axis_name="s")
    run = pl.kernel(
        _sc_body,
        out_type=(
            jax.ShapeDtypeStruct((N * CPAD,), jnp.float32),
            jax.ShapeDtypeStruct((N,), jnp.float32),
            jax.ShapeDtypeStruct((N,), jnp.float32),
            jax.ShapeDtypeStruct((N,), jnp.float32),
            jax.ShapeDtypeStruct((N,), jnp.int32),
        ),
        mesh=mesh,
        compiler_params=pltpu.CompilerParams(needs_layout_passes=False),
        scratch_types=[
            pltpu.VMEM((NCHUNK, 128), jnp.int32),
            pltpu.VMEM((NDET * CPAD,), jnp.float32),
            pltpu.VMEM((NDET,), jnp.int32),
            pltpu.VMEM((NDET,), jnp.int32),
            pltpu.VMEM((NDET,), jnp.int32),
            pltpu.VMEM((NDET,), jnp.int32),
            pltpu.VMEM((NDET,), jnp.float32),
            pltpu.VMEM((B,), jnp.int32),
            pltpu.VMEM((NDET,), jnp.int32),
            pltpu.VMEM((NDET,), jnp.float32),
            pltpu.VMEM((NDET,), jnp.float32),
            pltpu.SemaphoreType.DMA,
            pltpu.SemaphoreType.DMA,
        ],
    )
    params_pad, conf, px, py, reorg = run(
        pm_flat, cm_flat, batch_ids, flat_inds, meta_batch_ids)

    params_pred = params_pad.reshape(N, CPAD)[:, :C]
    center_preds = jnp.stack([px, py], axis=1)
    center_confs = conf.reshape(N, 1)
    return params_pred, center_preds, center_confs, reorg


# speedup vs baseline: 4.5369x; 1.0187x over previous
import jax
import jax.numpy as jnp
from jax import lax
from jax.experimental import pallas as pl
from jax.experimental.pallas import tpu as pltpu
from jax.experimental.pallas import tpu_sc as plsc

B = 16
C = 145
H = 128
W = 128
HW = H * W
N = 2048
NW = 32
NDET = N // NW
NELEM = NDET * C
NCHUNK = (NELEM + 127) // 128
LANES = 16


def _sc_body(pm_hbm, cm_hbm, bid_hbm, hw_hbm, meta_hbm,
             out_params, out_conf, out_preds, out_reorg,
             idx2d, buf, bids_v, hw_v, base_v, cidx_v, conf_buf,
             meta_v, reorg_buf, preds_buf, sem, sem2):
    wid = lax.axis_index("s") * 2 + lax.axis_index("c")
    det0 = wid * NDET

    pltpu.sync_copy(bid_hbm.at[pl.ds(det0, NDET)], bids_v)
    pltpu.sync_copy(hw_hbm.at[pl.ds(det0, NDET)], hw_v)
    pltpu.sync_copy(meta_hbm, meta_v)

    for t in range(NDET // LANES):
        sl = pl.ds(t * LANES, LANES)
        b = bids_v[sl]
        hw = hw_v[sl]
        base_v[sl] = b * (C * HW) + hw
        cidx_v[sl] = b * HW + hw
        reorg_buf[sl] = plsc.load_gather(meta_v, [b])
        px = (hw & (W - 1)).astype(jnp.float32) * 4.0
        py = lax.shift_right_logical(hw, 7).astype(jnp.float32) * 4.0
        pos = lax.iota(jnp.int32, LANES) * 2 + t * 2 * LANES
        plsc.store_scatter(preds_buf, [pos], px)
        plsc.store_scatter(preds_buf, [pos + 1], py)

    def gen_fire(j, _):
        for v in range(8):
            p0 = pl.multiple_of(j * 128, 128) + v * LANES
            p = p0 + lax.iota(jnp.int32, LANES)
            p = jnp.minimum(p, NELEM - 1)
            n_loc = p // C
            c = p - n_loc * C
            bse = plsc.load_gather(base_v, [n_loc])
            idx2d[j, pl.ds(v * LANES, LANES)] = bse + c * HW
        pltpu.async_copy(pm_hbm.at[idx2d.at[j]],
                         buf.at[pl.ds(pl.multiple_of(j * 128, 128), 128)],
                         sem)
        return 0

    lax.fori_loop(0, NCHUNK, gen_fire, 0)

    pltpu.async_copy(cm_hbm.at[cidx_v], conf_buf, sem2).wait()
    pltpu.sync_copy(conf_buf, out_conf.at[pl.ds(det0, NDET)])
    pltpu.sync_copy(preds_buf, out_preds.at[pl.ds(det0 * 2, NDET * 2)])
    pltpu.sync_copy(reorg_buf, out_reorg.at[pl.ds(det0, NDET)])

    def drain(j, _):
        pltpu.make_async_copy(pm_hbm.at[idx2d.at[j]],
                              buf.at[pl.ds(pl.multiple_of(j * 128, 128), 128)],
                              sem).wait()
        return 0

    lax.fori_loop(0, NCHUNK, drain, 0)

    pltpu.sync_copy(buf.at[pl.ds(0, NELEM)],
                    out_params.at[pl.ds(det0 * C, NELEM)])


@jax.jit
def kernel(params_maps, center_map, batch_ids, flat_inds, meta_batch_ids):
    pm_flat = params_maps.reshape(-1)
    cm_flat = center_map.reshape(-1)

    mesh = plsc.VectorSubcoreMesh(core_axis_name="c", subcore_axis_name="s")
    run = pl.kernel(
        _sc_body,
        out_type=(
            jax.ShapeDtypeStruct((N * C,), jnp.float32),
            jax.ShapeDtypeStruct((N,), jnp.float32),
            jax.ShapeDtypeStruct((N * 2,), jnp.float32),
            jax.ShapeDtypeStruct((N,), jnp.int32),
        ),
        mesh=mesh,
        compiler_params=pltpu.CompilerParams(needs_layout_passes=False),
        scratch_types=[
            pltpu.VMEM((NCHUNK, 128), jnp.int32),
            pltpu.VMEM((NCHUNK * 128,), jnp.float32),
            pltpu.VMEM((NDET,), jnp.int32),
            pltpu.VMEM((NDET,), jnp.int32),
            pltpu.VMEM((NDET,), jnp.int32),
            pltpu.VMEM((NDET,), jnp.int32),
            pltpu.VMEM((NDET,), jnp.float32),
            pltpu.VMEM((B,), jnp.int32),
            pltpu.VMEM((NDET,), jnp.int32),
            pltpu.VMEM((NDET * 2,), jnp.float32),
            pltpu.SemaphoreType.DMA,
            pltpu.SemaphoreType.DMA,
        ],
    )
    params_flat, conf, preds, reorg = run(
        pm_flat, cm_flat, batch_ids, flat_inds, meta_batch_ids)

    params_pred = params_flat.reshape(N, C)
    center_preds = preds.reshape(N, 2)
    center_confs = conf.reshape(N, 1)
    return params_pred, center_preds, center_confs, reorg
